# baseline (speedup 1.0000x reference)
import functools
import jax
import jax.numpy as jnp
from jax import lax
from jax.experimental import pallas as pl
from jax.experimental.pallas import tpu as pltpu
from jax.experimental.pallas import tpu_sc as plsc

NC = 2
NS = 16
NW = NC * NS

D = 64
WPA = 6
AV_PAD = 1024
APW = AV_PAD // NW
IPW = APW * WPA

_mesh = plsc.VectorSubcoreMesh(core_axis_name="c", subcore_axis_name="s")
_params = pltpu.CompilerParams(use_tc_tiling_on_sc=False)


def _wid():
    return lax.axis_index("s") * NC + lax.axis_index("c")


@functools.partial(
    pl.kernel,
    out_type=jax.ShapeDtypeStruct((AV_PAD, D), jnp.float32),
    mesh=_mesh,
    scratch_types=[
        pltpu.VMEM((IPW,), jnp.int32),
        pltpu.VMEM((IPW, D), jnp.float32),
        pltpu.VMEM((APW, D), jnp.float32),
        pltpu.SemaphoreType.DMA,
    ],
    compiler_params=_params,
)
def _build_table(a2w_hbm, emb_hbm, table_hbm, idx_v, rows_v, out_v, sem):
    wid = _wid()
    base = wid * IPW
    pltpu.sync_copy(a2w_hbm.at[pl.ds(base, IPW)], idx_v)
    h = IPW // 2
    pltpu.async_copy(emb_hbm.at[idx_v.at[pl.ds(0, h)]],
                     rows_v.at[pl.ds(0, h)], sem).wait()
    pltpu.async_copy(emb_hbm.at[idx_v.at[pl.ds(h, h)]],
                     rows_v.at[pl.ds(h, h)], sem).wait()
    for j in range(APW):
        for c in range(D // 16):
            s = pl.ds(16 * c, 16)
            acc = rows_v[WPA * j, s]
            for k in range(1, WPA):
                acc = acc + rows_v[WPA * j + k, s]
            out_v[j, s] = acc
    pltpu.sync_copy(out_v, table_hbm.at[pl.ds(wid * APW, APW)])


BATCH = 16384
SEQ = 50
BPW = BATCH // NW
NBUF = 8
NGRP = BPW // NBUF


@functools.partial(
    pl.kernel,
    out_type=jax.ShapeDtypeStruct((BATCH, SEQ, D), jnp.float32),
    mesh=_mesh,
    scratch_types=[
        pltpu.VMEM((BPW, SEQ), jnp.int32),
        [pltpu.VMEM((SEQ, D), jnp.float32) for _ in range(NBUF)],
        pltpu.SemaphoreType.DMA,
        pltpu.SemaphoreType.DMA,
    ],
    compiler_params=_params,
)
def _lookup(x_hbm, table_hbm, out_hbm, idx_v, bufs, gsem, osem):
    wid = _wid()
    base = wid * BPW
    pltpu.sync_copy(x_hbm.at[pl.ds(base, BPW), :], idx_v)

    def body(g, carry):
        r0 = g * NBUF

        @pl.when(g > 0)
        def _():
            for b in range(NBUF):
                pltpu.make_async_copy(bufs[b], out_hbm.at[base], osem).wait()

        for b in range(NBUF):
            pltpu.async_copy(table_hbm.at[idx_v.at[r0 + b]], bufs[b], gsem)
        for b in range(NBUF):
            pltpu.make_async_copy(
                table_hbm.at[idx_v.at[0]], bufs[b], gsem).wait()
        for b in range(NBUF):
            pltpu.async_copy(bufs[b], out_hbm.at[base + r0 + b], osem)
        return carry

    lax.fori_loop(0, NGRP, body, 0)
    for b in range(NBUF):
        pltpu.make_async_copy(bufs[b], out_hbm.at[base], osem).wait()


def kernel(x, action_to_words, word_embedding):
    a2w_flat = jnp.pad(action_to_words.reshape(-1),
                       (0, AV_PAD * WPA - action_to_words.size))
    table = _build_table(a2w_flat, word_embedding)
    return _lookup(x, table)


# speedup vs baseline: 1.0003x; 1.0003x over previous
import functools
import jax
import jax.numpy as jnp
from jax import lax
from jax.experimental import pallas as pl
from jax.experimental.pallas import tpu as pltpu
from jax.experimental.pallas import tpu_sc as plsc

NC = 2
NS = 16
NW = NC * NS

D = 64
WPA = 6
AV_PAD = 1024
APW = AV_PAD // NW
IPW = APW * WPA

_mesh = plsc.VectorSubcoreMesh(core_axis_name="c", subcore_axis_name="s")
_params = pltpu.CompilerParams(use_tc_tiling_on_sc=False)


def _wid():
    return lax.axis_index("s") * NC + lax.axis_index("c")


@functools.partial(
    pl.kernel,
    out_type=jax.ShapeDtypeStruct((AV_PAD, D), jnp.float32),
    mesh=_mesh,
    scratch_types=[
        pltpu.VMEM((IPW,), jnp.int32),
        pltpu.VMEM((IPW, D), jnp.float32),
        pltpu.VMEM((APW, D), jnp.float32),
        pltpu.SemaphoreType.DMA,
    ],
    compiler_params=_params,
)
def _build_table(a2w_hbm, emb_hbm, table_hbm, idx_v, rows_v, out_v, sem):
    wid = _wid()
    base = wid * IPW
    pltpu.sync_copy(a2w_hbm.at[pl.ds(base, IPW)], idx_v)
    h = IPW // 2
    pltpu.async_copy(emb_hbm.at[idx_v.at[pl.ds(0, h)]],
                     rows_v.at[pl.ds(0, h)], sem).wait()
    pltpu.async_copy(emb_hbm.at[idx_v.at[pl.ds(h, h)]],
                     rows_v.at[pl.ds(h, h)], sem).wait()
    for j in range(APW):
        for c in range(D // 16):
            s = pl.ds(16 * c, 16)
            acc = rows_v[WPA * j, s]
            for k in range(1, WPA):
                acc = acc + rows_v[WPA * j + k, s]
            out_v[j, s] = acc
    pltpu.sync_copy(out_v, table_hbm.at[pl.ds(wid * APW, APW)])


BATCH = 16384
SEQ = 50
BPW = BATCH // NW
NBUF = 8
NGRP = BPW // NBUF


@functools.partial(
    pl.kernel,
    out_type=jax.ShapeDtypeStruct((BATCH * SEQ, D), jnp.float32),
    mesh=_mesh,
    scratch_types=[
        pltpu.VMEM((BPW, SEQ), jnp.int32),
        [pltpu.VMEM((SEQ, D), jnp.float32) for _ in range(NBUF)],
        pltpu.SemaphoreType.DMA,
        pltpu.SemaphoreType.DMA,
    ],
    compiler_params=_params,
)
def _lookup(x_hbm, table_hbm, out_hbm, idx_v, bufs, gsem, osem):
    wid = _wid()
    base = wid * BPW
    pltpu.sync_copy(x_hbm.at[pl.ds(base, BPW), :], idx_v)

    def body(g, carry):
        r0 = g * NBUF

        @pl.when(g > 0)
        def _():
            for b in range(NBUF):
                pltpu.make_async_copy(
                    bufs[b], out_hbm.at[pl.ds(0, SEQ)], osem).wait()

        for b in range(NBUF):
            pltpu.async_copy(table_hbm.at[idx_v.at[r0 + b]], bufs[b], gsem)
        for b in range(NBUF):
            pltpu.make_async_copy(
                table_hbm.at[idx_v.at[0]], bufs[b], gsem).wait()
        for b in range(NBUF):
            pltpu.async_copy(
                bufs[b], out_hbm.at[pl.ds((base + r0 + b) * SEQ, SEQ)], osem)
        return carry

    lax.fori_loop(0, NGRP, body, 0)
    for b in range(NBUF):
        pltpu.make_async_copy(bufs[b], out_hbm.at[pl.ds(0, SEQ)], osem).wait()


def kernel(x, action_to_words, word_embedding):
    b, s = x.shape
    a2w_flat = jnp.pad(action_to_words.reshape(-1),
                       (0, AV_PAD * WPA - action_to_words.size))
    table = _build_table(a2w_flat, word_embedding)
    return _lookup(x, table).reshape(b, s, D)
